# baseline (speedup 1.0000x reference)
import jax
import jax.numpy as jnp
from jax.experimental import pallas as pl
from jax.experimental.pallas import tpu as pltpu

DIM = 256
AH = 4
DH = 64
NTOK = 65
NBLK = 6
NCLS = 100
CHUNK = 16

_DNT = (((1,), (1,)), ((), ()))
_F32 = jnp.float32


def _matmul_t(a, w):
    return jax.lax.dot_general(a, w, _DNT, preferred_element_type=_F32)


def _layernorm(v, g, b, eps=1e-5):
    m = jnp.mean(v, axis=-1, keepdims=True)
    c = v - m
    var = jnp.mean(c * c, axis=-1, keepdims=True)
    return c * jax.lax.rsqrt(var + eps) * g + b


def _fwd_kernel(*refs):
    (patches_ref, pw_ref, pb_ref, cls_ref, pos_ref) = refs[:5]
    blk = refs[5:5 + 12 * NBLK]
    (fng_ref, fnb_ref, hw_ref, hb_ref, aw_ref, ab_ref,
     logits_ref, aux_ref) = refs[5 + 12 * NBLK:]

    T = CHUNK * NTOK
    emb = _matmul_t(patches_ref[...], pw_ref[...]) + pb_ref[...]
    emb = emb.reshape(CHUNK, NTOK - 1, DIM) + pos_ref[...][None]
    cls = jnp.broadcast_to(cls_ref[...].reshape(1, 1, DIM), (CHUNK, 1, DIM))
    x = jnp.concatenate([cls, emb], axis=1).reshape(T, DIM)

    for i in range(NBLK):
        (n1g, n1b, inw, inb, outw, outb,
         n2g, n2b, w1, b1, w2, b2) = blk[12 * i:12 * i + 12]
        h = _layernorm(x, n1g[...], n1b[...])
        qkv = _matmul_t(h, inw[...]) + inb[...]
        q = qkv[:, 0:DIM].reshape(CHUNK, NTOK, DIM)
        k = qkv[:, DIM:2 * DIM].reshape(CHUNK, NTOK, DIM)
        v = qkv[:, 2 * DIM:3 * DIM].reshape(CHUNK, NTOK, DIM)
        heads = []
        for hh in range(AH):
            sl = slice(hh * DH, (hh + 1) * DH)
            qh, kh, vh = q[:, :, sl], k[:, :, sl], v[:, :, sl]
            att = jax.lax.dot_general(
                qh, kh, (((2,), (2,)), ((0,), (0,))),
                preferred_element_type=_F32) * (1.0 / 8.0)
            att = jnp.exp(att - jnp.max(att, axis=-1, keepdims=True))
            att = att / jnp.sum(att, axis=-1, keepdims=True)
            heads.append(jax.lax.dot_general(
                att, vh, (((2,), (1,)), ((0,), (0,))),
                preferred_element_type=_F32))
        o = jnp.concatenate(heads, axis=-1).reshape(T, DIM)
        x = x + _matmul_t(o, outw[...]) + outb[...]

        h2 = _layernorm(x, n2g[...], n2b[...])
        g1 = _matmul_t(h2, w1[...]) + b1[...]
        g1 = 0.5 * g1 * (1.0 + jax.lax.erf(g1 * 0.7071067811865476))
        x = x + _matmul_t(g1, w2[...]) + b2[...]

        if i == 3:
            clstok = x.reshape(CHUNK, NTOK, DIM)[:, 0, :]
            aux_ref[...] = _matmul_t(clstok, aw_ref[...]) + ab_ref[...]

    clstok = x.reshape(CHUNK, NTOK, DIM)[:, 0, :]
    hc = _layernorm(clstok, fng_ref[...], fnb_ref[...])
    logits_ref[...] = _matmul_t(hc, hw_ref[...]) + hb_ref[...]


def _full(a):
    nd = a.ndim
    return pl.BlockSpec(a.shape, lambda i, _n=nd: (0,) * _n)


def kernel(x, params):
    B = x.shape[0]
    p = params
    patches = x.reshape(B, 3, 8, 4, 8, 4).transpose(0, 2, 4, 1, 3, 5)
    patches = patches.reshape(B * (NTOK - 1), 48)
    args = [
        patches,
        p['patch_w'].reshape(DIM, 48),
        p['patch_b'].reshape(1, DIM),
        (p['cls_token'][0, 0] + p['pos_embed'][0, 0]).reshape(1, DIM),
        p['pos_embed'][0, 1:NTOK],
    ]
    for bp in p['blocks']:
        e0 = bp['moe']['experts'][0]
        args += [
            bp['n1_g'].reshape(1, DIM), bp['n1_b'].reshape(1, DIM),
            bp['attn']['in_w'], bp['attn']['in_b'].reshape(1, 3 * DIM),
            bp['attn']['out_w'], bp['attn']['out_b'].reshape(1, DIM),
            bp['n2_g'].reshape(1, DIM), bp['n2_b'].reshape(1, DIM),
            e0['w1'], e0['b1'].reshape(1, 2 * DIM),
            e0['w2'], e0['b2'].reshape(1, DIM),
        ]
    args += [
        p['fn_g'].reshape(1, DIM), p['fn_b'].reshape(1, DIM),
        p['head_w'], p['head_b'].reshape(1, NCLS),
        p['aux_w'], p['aux_b'].reshape(1, NCLS),
    ]
    nsteps = B // CHUNK
    in_specs = [pl.BlockSpec((CHUNK * (NTOK - 1), 48), lambda i: (i, 0))]
    in_specs += [_full(a) for a in args[1:]]
    out_spec = pl.BlockSpec((CHUNK, NCLS), lambda i: (i, 0))
    logits, aux = pl.pallas_call(
        _fwd_kernel,
        grid=(nsteps,),
        in_specs=in_specs,
        out_specs=(out_spec, out_spec),
        out_shape=(
            jax.ShapeDtypeStruct((B, NCLS), _F32),
            jax.ShapeDtypeStruct((B, NCLS), _F32),
        ),
        compiler_params=pltpu.CompilerParams(
            dimension_semantics=("parallel",),
            vmem_limit_bytes=60 * 1024 * 1024),
    )(*args)
    return logits, aux


# speedup vs baseline: 1.0071x; 1.0071x over previous
import jax
import jax.numpy as jnp
from jax.experimental import pallas as pl
from jax.experimental.pallas import tpu as pltpu

DIM = 256
AH = 4
DH = 64
NTOK = 65
NBLK = 6
NCLS = 100
CHUNK = 16

_DNT = (((1,), (1,)), ((), ()))
_F32 = jnp.float32
_BF16 = jnp.bfloat16


def _matmul_t(a, w):
    return jax.lax.dot_general(a.astype(_BF16), w, _DNT,
                               preferred_element_type=_F32)


def _layernorm(v, g, b, eps=1e-5):
    m = jnp.mean(v, axis=-1, keepdims=True)
    c = v - m
    var = jnp.mean(c * c, axis=-1, keepdims=True)
    return c * jax.lax.rsqrt(var + eps) * g + b


def _fwd_kernel(*refs):
    (patches_ref, pw_ref, pb_ref, cls_ref, pos_ref) = refs[:5]
    blk = refs[5:5 + 12 * NBLK]
    (fng_ref, fnb_ref, hw_ref, hb_ref, aw_ref, ab_ref,
     logits_ref, aux_ref) = refs[5 + 12 * NBLK:]

    T = CHUNK * NTOK
    emb = _matmul_t(patches_ref[...], pw_ref[...]) + pb_ref[...]
    emb = emb.reshape(CHUNK, NTOK - 1, DIM) + pos_ref[...][None]
    cls = jnp.broadcast_to(cls_ref[...].reshape(1, 1, DIM), (CHUNK, 1, DIM))
    x = jnp.concatenate([cls, emb], axis=1).reshape(T, DIM)

    for i in range(NBLK):
        (n1g, n1b, inw, inb, outw, outb,
         n2g, n2b, w1, b1, w2, b2) = blk[12 * i:12 * i + 12]
        h = _layernorm(x, n1g[...], n1b[...])
        qkv = _matmul_t(h, inw[...]) + inb[...]
        q = qkv[:, 0:DIM].reshape(CHUNK, NTOK, DIM)
        k = qkv[:, DIM:2 * DIM].reshape(CHUNK, NTOK, DIM)
        v = qkv[:, 2 * DIM:3 * DIM].reshape(CHUNK, NTOK, DIM)
        heads = []
        for hh in range(AH):
            sl = slice(hh * DH, (hh + 1) * DH)
            qh = q[:, :, sl].astype(_BF16)
            kh = k[:, :, sl].astype(_BF16)
            vh = v[:, :, sl].astype(_BF16)
            att = jax.lax.dot_general(
                qh, kh, (((2,), (2,)), ((0,), (0,))),
                preferred_element_type=_F32) * (1.0 / 8.0)
            att = jnp.exp(att - jnp.max(att, axis=-1, keepdims=True))
            att = att / jnp.sum(att, axis=-1, keepdims=True)
            heads.append(jax.lax.dot_general(
                att.astype(_BF16), vh, (((2,), (1,)), ((0,), (0,))),
                preferred_element_type=_F32))
        o = jnp.concatenate(heads, axis=-1).reshape(T, DIM)
        x = x + _matmul_t(o, outw[...]) + outb[...]

        h2 = _layernorm(x, n2g[...], n2b[...])
        g1 = _matmul_t(h2, w1[...]) + b1[...]
        g1 = 0.5 * g1 * (1.0 + jax.lax.erf(g1 * 0.7071067811865476))
        x = x + _matmul_t(g1, w2[...]) + b2[...]

        if i == 3:
            clstok = x.reshape(CHUNK, NTOK, DIM)[:, 0, :]
            aux_ref[...] = _matmul_t(clstok, aw_ref[...]) + ab_ref[...]

    clstok = x.reshape(CHUNK, NTOK, DIM)[:, 0, :]
    hc = _layernorm(clstok, fng_ref[...], fnb_ref[...])
    logits_ref[...] = _matmul_t(hc, hw_ref[...]) + hb_ref[...]


def _full(a):
    nd = a.ndim
    return pl.BlockSpec(a.shape, lambda i, _n=nd: (0,) * _n)


def kernel(x, params):
    B = x.shape[0]
    p = params
    patches = x.reshape(B, 3, 8, 4, 8, 4).transpose(0, 2, 4, 1, 3, 5)
    patches = patches.reshape(B * (NTOK - 1), 48)
    args = [
        patches,
        p['patch_w'].reshape(DIM, 48).astype(_BF16),
        p['patch_b'].reshape(1, DIM),
        (p['cls_token'][0, 0] + p['pos_embed'][0, 0]).reshape(1, DIM),
        p['pos_embed'][0, 1:NTOK],
    ]
    for bp in p['blocks']:
        e0 = bp['moe']['experts'][0]
        args += [
            bp['n1_g'].reshape(1, DIM), bp['n1_b'].reshape(1, DIM),
            bp['attn']['in_w'].astype(_BF16), bp['attn']['in_b'].reshape(1, 3 * DIM),
            bp['attn']['out_w'].astype(_BF16), bp['attn']['out_b'].reshape(1, DIM),
            bp['n2_g'].reshape(1, DIM), bp['n2_b'].reshape(1, DIM),
            e0['w1'].astype(_BF16), e0['b1'].reshape(1, 2 * DIM),
            e0['w2'].astype(_BF16), e0['b2'].reshape(1, DIM),
        ]
    args += [
        p['fn_g'].reshape(1, DIM), p['fn_b'].reshape(1, DIM),
        p['head_w'].astype(_BF16), p['head_b'].reshape(1, NCLS),
        p['aux_w'].astype(_BF16), p['aux_b'].reshape(1, NCLS),
    ]
    nsteps = B // CHUNK
    in_specs = [pl.BlockSpec((CHUNK * (NTOK - 1), 48), lambda i: (i, 0))]
    in_specs += [_full(a) for a in args[1:]]
    out_spec = pl.BlockSpec((CHUNK, NCLS), lambda i: (i, 0))
    logits, aux = pl.pallas_call(
        _fwd_kernel,
        grid=(nsteps,),
        in_specs=in_specs,
        out_specs=(out_spec, out_spec),
        out_shape=(
            jax.ShapeDtypeStruct((B, NCLS), _F32),
            jax.ShapeDtypeStruct((B, NCLS), _F32),
        ),
        compiler_params=pltpu.CompilerParams(
            dimension_semantics=("parallel",),
            vmem_limit_bytes=60 * 1024 * 1024),
    )(*args)
    return logits, aux


# speedup vs baseline: 1.0536x; 1.0462x over previous
import jax
import jax.numpy as jnp
from jax.experimental import pallas as pl
from jax.experimental.pallas import tpu as pltpu

DIM = 256
AH = 4
DH = 64
NTOK = 65
NBLK = 6
NCLS = 100
CHUNK = 32

_DNT = (((1,), (1,)), ((), ()))
_F32 = jnp.float32
_BF16 = jnp.bfloat16


def _matmul_t(a, w):
    return jax.lax.dot_general(a.astype(_BF16), w, _DNT,
                               preferred_element_type=_F32)


def _layernorm(v, g, b, eps=1e-5):
    m = jnp.mean(v, axis=-1, keepdims=True)
    c = v - m
    var = jnp.mean(c * c, axis=-1, keepdims=True)
    return c * jax.lax.rsqrt(var + eps) * g + b


def _fwd_kernel(*refs):
    (patches_ref, pw_ref, pb_ref, cls_ref, pos_ref) = refs[:5]
    blk = refs[5:5 + 12 * NBLK]
    (fng_ref, fnb_ref, hw_ref, hb_ref, aw_ref, ab_ref,
     logits_ref, aux_ref) = refs[5 + 12 * NBLK:]

    T = CHUNK * NTOK
    emb = _matmul_t(patches_ref[...], pw_ref[...]) + pb_ref[...]
    emb = emb.reshape(CHUNK, NTOK - 1, DIM) + pos_ref[...][None]
    cls = jnp.broadcast_to(cls_ref[...].reshape(1, 1, DIM), (CHUNK, 1, DIM))
    x = jnp.concatenate([cls, emb], axis=1).reshape(T, DIM)

    for i in range(NBLK):
        (n1g, n1b, inw, inb, outw, outb,
         n2g, n2b, w1, b1, w2, b2) = blk[12 * i:12 * i + 12]
        h = _layernorm(x, n1g[...], n1b[...])
        qkv = _matmul_t(h, inw[...]) + inb[...]
        q = qkv[:, 0:DIM].reshape(CHUNK, NTOK, DIM)
        k = qkv[:, DIM:2 * DIM].reshape(CHUNK, NTOK, DIM)
        v = qkv[:, 2 * DIM:3 * DIM].reshape(CHUNK, NTOK, DIM)
        heads = []
        for hh in range(AH):
            sl = slice(hh * DH, (hh + 1) * DH)
            qh = q[:, :, sl].astype(_BF16)
            kh = k[:, :, sl].astype(_BF16)
            vh = v[:, :, sl].astype(_BF16)
            att = jax.lax.dot_general(
                qh, kh, (((2,), (2,)), ((0,), (0,))),
                preferred_element_type=_F32) * (1.0 / 8.0)
            att = jnp.exp(att - jnp.max(att, axis=-1, keepdims=True))
            att = att / jnp.sum(att, axis=-1, keepdims=True)
            heads.append(jax.lax.dot_general(
                att.astype(_BF16), vh, (((2,), (1,)), ((0,), (0,))),
                preferred_element_type=_F32))
        o = jnp.concatenate(heads, axis=-1).reshape(T, DIM)
        x = x + _matmul_t(o, outw[...]) + outb[...]

        h2 = _layernorm(x, n2g[...], n2b[...])
        g1 = _matmul_t(h2, w1[...]) + b1[...]
        g1 = 0.5 * g1 * (1.0 + jax.lax.erf(g1 * 0.7071067811865476))
        x = x + _matmul_t(g1, w2[...]) + b2[...]

        if i == 3:
            clstok = x.reshape(CHUNK, NTOK, DIM)[:, 0, :]
            aux_ref[...] = _matmul_t(clstok, aw_ref[...]) + ab_ref[...]

    clstok = x.reshape(CHUNK, NTOK, DIM)[:, 0, :]
    hc = _layernorm(clstok, fng_ref[...], fnb_ref[...])
    logits_ref[...] = _matmul_t(hc, hw_ref[...]) + hb_ref[...]


def _full(a):
    nd = a.ndim
    return pl.BlockSpec(a.shape, lambda i, _n=nd: (0,) * _n)


def kernel(x, params):
    B = x.shape[0]
    p = params
    patches = x.reshape(B, 3, 8, 4, 8, 4).transpose(0, 2, 4, 1, 3, 5)
    patches = patches.reshape(B * (NTOK - 1), 48)
    args = [
        patches,
        p['patch_w'].reshape(DIM, 48).astype(_BF16),
        p['patch_b'].reshape(1, DIM),
        (p['cls_token'][0, 0] + p['pos_embed'][0, 0]).reshape(1, DIM),
        p['pos_embed'][0, 1:NTOK],
    ]
    for bp in p['blocks']:
        e0 = bp['moe']['experts'][0]
        args += [
            bp['n1_g'].reshape(1, DIM), bp['n1_b'].reshape(1, DIM),
            bp['attn']['in_w'].astype(_BF16), bp['attn']['in_b'].reshape(1, 3 * DIM),
            bp['attn']['out_w'].astype(_BF16), bp['attn']['out_b'].reshape(1, DIM),
            bp['n2_g'].reshape(1, DIM), bp['n2_b'].reshape(1, DIM),
            e0['w1'].astype(_BF16), e0['b1'].reshape(1, 2 * DIM),
            e0['w2'].astype(_BF16), e0['b2'].reshape(1, DIM),
        ]
    args += [
        p['fn_g'].reshape(1, DIM), p['fn_b'].reshape(1, DIM),
        p['head_w'].astype(_BF16), p['head_b'].reshape(1, NCLS),
        p['aux_w'].astype(_BF16), p['aux_b'].reshape(1, NCLS),
    ]
    nsteps = B // CHUNK
    in_specs = [pl.BlockSpec((CHUNK * (NTOK - 1), 48), lambda i: (i, 0))]
    in_specs += [_full(a) for a in args[1:]]
    out_spec = pl.BlockSpec((CHUNK, NCLS), lambda i: (i, 0))
    logits, aux = pl.pallas_call(
        _fwd_kernel,
        grid=(nsteps,),
        in_specs=in_specs,
        out_specs=(out_spec, out_spec),
        out_shape=(
            jax.ShapeDtypeStruct((B, NCLS), _F32),
            jax.ShapeDtypeStruct((B, NCLS), _F32),
        ),
        compiler_params=pltpu.CompilerParams(
            dimension_semantics=("parallel",),
            vmem_limit_bytes=60 * 1024 * 1024),
    )(*args)
    return logits, aux


# speedup vs baseline: 1.3098x; 1.2431x over previous
import jax
import jax.numpy as jnp
from jax.experimental import pallas as pl
from jax.experimental.pallas import tpu as pltpu

DIM = 256
AH = 4
DH = 64
NTOK = 65
NBLK = 6
NCLS = 100
CHUNK = 64

_DNT = (((1,), (1,)), ((), ()))
_F32 = jnp.float32
_BF16 = jnp.bfloat16


def _matmul_t(a, w):
    return jax.lax.dot_general(a.astype(_BF16), w, _DNT,
                               preferred_element_type=_F32)


def _layernorm(v, g, b, eps=1e-5):
    m = jnp.mean(v, axis=-1, keepdims=True)
    c = v - m
    var = jnp.mean(c * c, axis=-1, keepdims=True)
    return c * jax.lax.rsqrt(var + eps) * g + b


def _fwd_kernel(*refs):
    (patches_ref, pw_ref, pb_ref, cls_ref, pos_ref) = refs[:5]
    blk = refs[5:5 + 12 * NBLK]
    (fng_ref, fnb_ref, hw_ref, hb_ref, aw_ref, ab_ref,
     logits_ref, aux_ref) = refs[5 + 12 * NBLK:]

    T = CHUNK * NTOK
    emb = _matmul_t(patches_ref[...], pw_ref[...]) + pb_ref[...]
    emb = emb.reshape(CHUNK, NTOK - 1, DIM) + pos_ref[...][None]
    cls = jnp.broadcast_to(cls_ref[...].reshape(1, 1, DIM), (CHUNK, 1, DIM))
    x = jnp.concatenate([cls, emb], axis=1).reshape(T, DIM)

    for i in range(NBLK):
        (n1g, n1b, inw, inb, outw, outb,
         n2g, n2b, w1, b1, w2, b2) = blk[12 * i:12 * i + 12]
        h = _layernorm(x, n1g[...], n1b[...])
        qkv = _matmul_t(h, inw[...]) + inb[...]
        q = qkv[:, 0:DIM].reshape(CHUNK, NTOK, DIM)
        k = qkv[:, DIM:2 * DIM].reshape(CHUNK, NTOK, DIM)
        v = qkv[:, 2 * DIM:3 * DIM].reshape(CHUNK, NTOK, DIM)
        heads = []
        for hh in range(AH):
            sl = slice(hh * DH, (hh + 1) * DH)
            qh = q[:, :, sl].astype(_BF16)
            kh = k[:, :, sl].astype(_BF16)
            vh = v[:, :, sl].astype(_BF16)
            att = jax.lax.dot_general(
                qh, kh, (((2,), (2,)), ((0,), (0,))),
                preferred_element_type=_F32) * (1.0 / 8.0)
            att = jnp.exp(att - jnp.max(att, axis=-1, keepdims=True))
            att = att / jnp.sum(att, axis=-1, keepdims=True)
            heads.append(jax.lax.dot_general(
                att.astype(_BF16), vh, (((2,), (1,)), ((0,), (0,))),
                preferred_element_type=_F32))
        o = jnp.concatenate(heads, axis=-1).reshape(T, DIM)
        x = x + _matmul_t(o, outw[...]) + outb[...]

        h2 = _layernorm(x, n2g[...], n2b[...])
        g1 = _matmul_t(h2, w1[...]) + b1[...]
        g1 = 0.5 * g1 * (1.0 + jax.lax.erf(g1 * 0.7071067811865476))
        x = x + _matmul_t(g1, w2[...]) + b2[...]

        if i == 3:
            clstok = x.reshape(CHUNK, NTOK, DIM)[:, 0, :]
            aux_ref[...] = _matmul_t(clstok, aw_ref[...]) + ab_ref[...]

    clstok = x.reshape(CHUNK, NTOK, DIM)[:, 0, :]
    hc = _layernorm(clstok, fng_ref[...], fnb_ref[...])
    logits_ref[...] = _matmul_t(hc, hw_ref[...]) + hb_ref[...]


def _full(a):
    nd = a.ndim
    return pl.BlockSpec(a.shape, lambda i, _n=nd: (0,) * _n)


def kernel(x, params):
    B = x.shape[0]
    p = params
    patches = x.reshape(B, 3, 8, 4, 8, 4).transpose(0, 2, 4, 1, 3, 5)
    patches = patches.reshape(B * (NTOK - 1), 48)
    args = [
        patches,
        p['patch_w'].reshape(DIM, 48).astype(_BF16),
        p['patch_b'].reshape(1, DIM),
        (p['cls_token'][0, 0] + p['pos_embed'][0, 0]).reshape(1, DIM),
        p['pos_embed'][0, 1:NTOK],
    ]
    for bp in p['blocks']:
        e0 = bp['moe']['experts'][0]
        args += [
            bp['n1_g'].reshape(1, DIM), bp['n1_b'].reshape(1, DIM),
            bp['attn']['in_w'].astype(_BF16), bp['attn']['in_b'].reshape(1, 3 * DIM),
            bp['attn']['out_w'].astype(_BF16), bp['attn']['out_b'].reshape(1, DIM),
            bp['n2_g'].reshape(1, DIM), bp['n2_b'].reshape(1, DIM),
            e0['w1'].astype(_BF16), e0['b1'].reshape(1, 2 * DIM),
            e0['w2'].astype(_BF16), e0['b2'].reshape(1, DIM),
        ]
    args += [
        p['fn_g'].reshape(1, DIM), p['fn_b'].reshape(1, DIM),
        p['head_w'].astype(_BF16), p['head_b'].reshape(1, NCLS),
        p['aux_w'].astype(_BF16), p['aux_b'].reshape(1, NCLS),
    ]
    nsteps = B // CHUNK
    in_specs = [pl.BlockSpec((CHUNK * (NTOK - 1), 48), lambda i: (i, 0))]
    in_specs += [_full(a) for a in args[1:]]
    out_spec = pl.BlockSpec((CHUNK, NCLS), lambda i: (i, 0))
    logits, aux = pl.pallas_call(
        _fwd_kernel,
        grid=(nsteps,),
        in_specs=in_specs,
        out_specs=(out_spec, out_spec),
        out_shape=(
            jax.ShapeDtypeStruct((B, NCLS), _F32),
            jax.ShapeDtypeStruct((B, NCLS), _F32),
        ),
        compiler_params=pltpu.CompilerParams(
            dimension_semantics=("parallel",),
            vmem_limit_bytes=60 * 1024 * 1024),
    )(*args)
    return logits, aux


# speedup vs baseline: 1.3435x; 1.0258x over previous
import jax
import jax.numpy as jnp
from jax.experimental import pallas as pl
from jax.experimental.pallas import tpu as pltpu

DIM = 256
AH = 4
DH = 64
NTOK = 65
NBLK = 6
NCLS = 100
CHUNK = 64

_DNT = (((1,), (1,)), ((), ()))
_F32 = jnp.float32
_BF16 = jnp.bfloat16


def _matmul_t(a, w):
    return jax.lax.dot_general(a.astype(_BF16), w, _DNT,
                               preferred_element_type=_F32)


def _norm(v, eps=1e-5):
    m = jnp.mean(v, axis=-1, keepdims=True)
    c = v - m
    var = jnp.mean(c * c, axis=-1, keepdims=True)
    return c * jax.lax.rsqrt(var + eps)


def _fwd_kernel(*refs):
    (patches_ref, pw_ref, cls_ref, pos_ref) = refs[:4]
    blk = refs[4:4 + 8 * NBLK]
    (hw_ref, hb_ref, aw_ref, ab_ref, logits_ref, aux_ref) = refs[4 + 8 * NBLK:]

    T = CHUNK * NTOK
    emb = _matmul_t(patches_ref[...], pw_ref[...])
    emb = emb.reshape(CHUNK, NTOK - 1, DIM) + pos_ref[...][None]
    cls = jnp.broadcast_to(cls_ref[...].reshape(1, 1, DIM), (CHUNK, 1, DIM))
    x = jnp.concatenate([cls, emb], axis=1).reshape(T, DIM)

    for i in range(NBLK):
        (inw, inb, outw, outb, w1, b1, w2, b2) = blk[8 * i:8 * i + 8]
        h = _norm(x)
        qkv = _matmul_t(h, inw[...]) + inb[...]
        q = qkv[:, 0:DIM].reshape(CHUNK, NTOK, DIM)
        k = qkv[:, DIM:2 * DIM].reshape(CHUNK, NTOK, DIM)
        v = qkv[:, 2 * DIM:3 * DIM].reshape(CHUNK, NTOK, DIM)
        heads = []
        for hh in range(AH):
            sl = slice(hh * DH, (hh + 1) * DH)
            qh = q[:, :, sl].astype(_BF16)
            kh = k[:, :, sl].astype(_BF16)
            vh = v[:, :, sl].astype(_BF16)
            att = jax.lax.dot_general(
                qh, kh, (((2,), (2,)), ((0,), (0,))),
                preferred_element_type=_F32)
            att = jnp.exp(att - jnp.max(att, axis=-1, keepdims=True))
            att = att / jnp.sum(att, axis=-1, keepdims=True)
            heads.append(jax.lax.dot_general(
                att.astype(_BF16), vh, (((2,), (1,)), ((0,), (0,))),
                preferred_element_type=_F32))
        o = jnp.concatenate(heads, axis=-1).reshape(T, DIM)
        x = x + _matmul_t(o, outw[...]) + outb[...]

        g1 = _matmul_t(_norm(x), w1[...]) + b1[...]
        g1 = 0.5 * g1 * (1.0 + jax.lax.erf(g1 * 0.7071067811865476))
        x = x + _matmul_t(g1, w2[...]) + b2[...]

        if i == 3:
            clstok = x.reshape(CHUNK, NTOK, DIM)[:, 0, :]
            aux_ref[...] = _matmul_t(clstok, aw_ref[...]) + ab_ref[...]

    clstok = x.reshape(CHUNK, NTOK, DIM)[:, 0, :]
    logits_ref[...] = _matmul_t(_norm(clstok), hw_ref[...]) + hb_ref[...]


def _full(a):
    nd = a.ndim
    return pl.BlockSpec(a.shape, lambda i, _n=nd: (0,) * _n)


def kernel(x, params):
    B = x.shape[0]
    p = params
    patches = x.reshape(B, 3, 8, 4, 8, 4).transpose(0, 2, 4, 1, 3, 5)
    patches = patches.reshape(B * (NTOK - 1), 48)
    qscale = jnp.concatenate(
        [jnp.full((DIM,), 1.0 / 8.0, _F32), jnp.ones((2 * DIM,), _F32)])
    args = [
        patches,
        p['patch_w'].reshape(DIM, 48).astype(_BF16),
        (p['cls_token'][0, 0] + p['pos_embed'][0, 0]).reshape(1, DIM),
        p['pos_embed'][0, 1:NTOK] + p['patch_b'][None, :],
    ]
    for bp in p['blocks']:
        e0 = bp['moe']['experts'][0]
        inw = bp['attn']['in_w'] * bp['n1_g'][None, :] * qscale[:, None]
        inb = (bp['attn']['in_w'] @ bp['n1_b'] + bp['attn']['in_b']) * qscale
        w1 = e0['w1'] * bp['n2_g'][None, :]
        b1 = e0['w1'] @ bp['n2_b'] + e0['b1']
        args += [
            inw.astype(_BF16), inb.reshape(1, 3 * DIM),
            bp['attn']['out_w'].astype(_BF16),
            bp['attn']['out_b'].reshape(1, DIM),
            w1.astype(_BF16), b1.reshape(1, 2 * DIM),
            e0['w2'].astype(_BF16), e0['b2'].reshape(1, DIM),
        ]
    hw = p['head_w'] * p['fn_g'][None, :]
    hb = p['head_w'] @ p['fn_b'] + p['head_b']
    args += [
        hw.astype(_BF16), hb.reshape(1, NCLS),
        p['aux_w'].astype(_BF16), p['aux_b'].reshape(1, NCLS),
    ]
    nsteps = B // CHUNK
    in_specs = [pl.BlockSpec((CHUNK * (NTOK - 1), 48), lambda i: (i, 0))]
    in_specs += [_full(a) for a in args[1:]]
    out_spec = pl.BlockSpec((CHUNK, NCLS), lambda i: (i, 0))
    logits, aux = pl.pallas_call(
        _fwd_kernel,
        grid=(nsteps,),
        in_specs=in_specs,
        out_specs=(out_spec, out_spec),
        out_shape=(
            jax.ShapeDtypeStruct((B, NCLS), _F32),
            jax.ShapeDtypeStruct((B, NCLS), _F32),
        ),
        compiler_params=pltpu.CompilerParams(
            dimension_semantics=("arbitrary",),
            vmem_limit_bytes=60 * 1024 * 1024),
    )(*args)
    return logits, aux


# speedup vs baseline: 1.3925x; 1.0364x over previous
import jax
import jax.numpy as jnp
from jax.experimental import pallas as pl
from jax.experimental.pallas import tpu as pltpu

DIM = 256
AH = 4
DH = 64
NTOK = 65
NBLK = 6
NCLS = 100
CHUNK = 64

_DNT = (((1,), (1,)), ((), ()))
_F32 = jnp.float32
_BF16 = jnp.bfloat16


def _matmul_t(a, w):
    return jax.lax.dot_general(a.astype(_BF16), w, _DNT,
                               preferred_element_type=_F32)


_DNJ = (((1,), (0,)), ((), ()))


def _norm(v, j, eps=1e-5):
    m = jax.lax.dot_general(v, j, _DNJ, preferred_element_type=_F32)[:, 0:1]
    ex2 = jax.lax.dot_general(v * v, j, _DNJ,
                              preferred_element_type=_F32)[:, 0:1]
    var = ex2 - m * m
    return (v - m) * jax.lax.rsqrt(var + eps)


def _fwd_kernel(*refs):
    (patches_ref, pw_ref, cls_ref, pos_ref, j_ref) = refs[:5]
    blk = refs[5:5 + 8 * NBLK]
    (hw_ref, hb_ref, aw_ref, ab_ref, logits_ref, aux_ref) = refs[5 + 8 * NBLK:]

    T = CHUNK * NTOK
    emb = _matmul_t(patches_ref[...], pw_ref[...])
    emb = emb.reshape(CHUNK, NTOK - 1, DIM) + pos_ref[...][None]
    cls = jnp.broadcast_to(cls_ref[...].reshape(1, 1, DIM), (CHUNK, 1, DIM))
    x = jnp.concatenate([cls, emb], axis=1).reshape(T, DIM)

    for i in range(NBLK):
        (inw, inb, outw, outb, w1, b1, w2, b2) = blk[8 * i:8 * i + 8]
        j = j_ref[...]
        h = _norm(x, j)
        qkv = _matmul_t(h, inw[...]) + inb[...]
        q = qkv[:, 0:DIM].reshape(CHUNK, NTOK, DIM)
        k = qkv[:, DIM:2 * DIM].reshape(CHUNK, NTOK, DIM)
        v = qkv[:, 2 * DIM:3 * DIM].reshape(CHUNK, NTOK, DIM)
        heads = []
        for hh in range(AH):
            sl = slice(hh * DH, (hh + 1) * DH)
            qh = q[:, :, sl].astype(_BF16)
            kh = k[:, :, sl].astype(_BF16)
            vh = v[:, :, sl].astype(_BF16)
            att = jax.lax.dot_general(
                qh, kh, (((2,), (2,)), ((0,), (0,))),
                preferred_element_type=_F32)
            att = jnp.exp(att - jnp.max(att, axis=-1, keepdims=True))
            att = att / jnp.sum(att, axis=-1, keepdims=True)
            heads.append(jax.lax.dot_general(
                att.astype(_BF16), vh, (((2,), (1,)), ((0,), (0,))),
                preferred_element_type=_F32))
        o = jnp.concatenate(heads, axis=-1).reshape(T, DIM)
        x = x + _matmul_t(o, outw[...]) + outb[...]

        g1 = _matmul_t(_norm(x, j), w1[...]) + b1[...]
        g1 = 0.5 * g1 * (1.0 + jax.lax.erf(g1 * 0.7071067811865476))
        x = x + _matmul_t(g1, w2[...]) + b2[...]

        if i == 3:
            clstok = x.reshape(CHUNK, NTOK, DIM)[:, 0, :]
            aux_ref[...] = _matmul_t(clstok, aw_ref[...]) + ab_ref[...]

    clstok = x.reshape(CHUNK, NTOK, DIM)[:, 0, :]
    logits_ref[...] = _matmul_t(_norm(clstok, j_ref[...]), hw_ref[...]) + hb_ref[...]


def _full(a):
    nd = a.ndim
    return pl.BlockSpec(a.shape, lambda i, _n=nd: (0,) * _n)


def kernel(x, params):
    B = x.shape[0]
    p = params
    patches = x.reshape(B, 3, 8, 4, 8, 4).transpose(0, 2, 4, 1, 3, 5)
    patches = patches.reshape(B * (NTOK - 1), 48)
    qscale = jnp.concatenate(
        [jnp.full((DIM,), 1.0 / 8.0, _F32), jnp.ones((2 * DIM,), _F32)])
    args = [
        patches,
        p['patch_w'].reshape(DIM, 48).astype(_BF16),
        (p['cls_token'][0, 0] + p['pos_embed'][0, 0]).reshape(1, DIM),
        p['pos_embed'][0, 1:NTOK] + p['patch_b'][None, :],
        jnp.zeros((DIM, 8), _F32).at[:, 0].set(1.0 / DIM),
    ]
    for bp in p['blocks']:
        e0 = bp['moe']['experts'][0]
        inw = bp['attn']['in_w'] * bp['n1_g'][None, :] * qscale[:, None]
        inb = (bp['attn']['in_w'] @ bp['n1_b'] + bp['attn']['in_b']) * qscale
        w1 = e0['w1'] * bp['n2_g'][None, :]
        b1 = e0['w1'] @ bp['n2_b'] + e0['b1']
        args += [
            inw.astype(_BF16), inb.reshape(1, 3 * DIM),
            bp['attn']['out_w'].astype(_BF16),
            bp['attn']['out_b'].reshape(1, DIM),
            w1.astype(_BF16), b1.reshape(1, 2 * DIM),
            e0['w2'].astype(_BF16), e0['b2'].reshape(1, DIM),
        ]
    hw = p['head_w'] * p['fn_g'][None, :]
    hb = p['head_w'] @ p['fn_b'] + p['head_b']
    args += [
        hw.astype(_BF16), hb.reshape(1, NCLS),
        p['aux_w'].astype(_BF16), p['aux_b'].reshape(1, NCLS),
    ]
    nsteps = B // CHUNK
    in_specs = [pl.BlockSpec((CHUNK * (NTOK - 1), 48), lambda i: (i, 0))]
    in_specs += [_full(a) for a in args[1:]]
    out_spec = pl.BlockSpec((CHUNK, NCLS), lambda i: (i, 0))
    logits, aux = pl.pallas_call(
        _fwd_kernel,
        grid=(nsteps,),
        in_specs=in_specs,
        out_specs=(out_spec, out_spec),
        out_shape=(
            jax.ShapeDtypeStruct((B, NCLS), _F32),
            jax.ShapeDtypeStruct((B, NCLS), _F32),
        ),
        compiler_params=pltpu.CompilerParams(
            dimension_semantics=("arbitrary",),
            vmem_limit_bytes=60 * 1024 * 1024),
    )(*args)
    return logits, aux
